# baseline (speedup 1.0000x reference)
import numpy as np
import jax
import jax.numpy as jnp
from jax.experimental import pallas as pl
from jax.experimental.pallas import tpu as pltpu

_B, _T, _C, _P, _D, _DF, _H, _DH = 8, 1024, 8, 16, 128, 256, 8, 16
_EL, _N, _L, _PRED, _TOPP = 2, 64, 512, 96, 0.5


def _pos_encoding():
    pos = np.arange(_L)[:, None].astype(np.float32)
    div = np.exp(np.arange(0, _D, 2).astype(np.float32) * (-np.log(10000.0) / _D))
    pe = np.zeros((_L, _D), dtype=np.float32)
    pe[:, 0::2] = np.sin(pos * div)
    pe[:, 1::2] = np.cos(pos * div)
    return pe


def _layer_body(x, wq, wk, wv, wo, rw, rb, g1, bg1, w1, b1, w2, b2, g2, bg2,
                loga_sel):
    rlog = jnp.dot(x, rw, preferred_element_type=jnp.float32) + rb
    l0, l1, l2 = rlog[:, 0:1], rlog[:, 1:2], rlog[:, 2:3]
    m = jnp.maximum(jnp.maximum(l0, l1), l2)
    e0, e1, e2 = jnp.exp(l0 - m), jnp.exp(l1 - m), jnp.exp(l2 - m)
    s = e0 + e1 + e2
    p0, p1, p2 = e0 / s, e1 / s, e2 / s
    cb0 = p1 * (p1 > p0) + p2 * (p2 > p0)
    cb1 = p0 * (p0 >= p1) + p2 * (p2 > p1)
    cb2 = p0 * (p0 >= p2) + p1 * (p1 >= p2)
    w0 = p0 * (cb0 < _TOPP)
    w1_ = p1 * (cb1 < _TOPP)
    w2_ = p2 * (cb2 < _TOPP)
    ws = w0 + w1_ + w2_ + 1e-9
    lw0 = jnp.log(w0 / ws)
    lw1 = jnp.log(w1_ / ws)
    lw2 = jnp.log(w2_ / ws)

    diag, same_s, same_t = loga_sel
    loga = jnp.where(diag, 0.0, jnp.where(same_s, lw0,
                     jnp.where(same_t, lw1, lw2)))

    q = jnp.dot(x, wq, preferred_element_type=jnp.float32)
    q = q * (1.0 / (_DH ** 0.5))
    k = jnp.dot(x, wk, preferred_element_type=jnp.float32)
    v = jnp.dot(x, wv, preferred_element_type=jnp.float32)
    ones_col = jnp.ones((_L, 1), jnp.float32)

    heads = []
    for h in range(_H):
        sl = slice(h * _DH, (h + 1) * _DH)
        logits = jax.lax.dot_general(
            q[:, sl], k[:, sl], (((1,), (1,)), ((), ())),
            preferred_element_type=jnp.float32) + loga
        mh = jnp.max(logits, axis=1, keepdims=True)
        ae = jnp.exp(logits - mh)
        vhe = jnp.concatenate([v[:, sl], ones_col], axis=1)
        nz = jnp.dot(ae, vhe, preferred_element_type=jnp.float32)
        heads.append(nz[:, :_DH] / (nz[:, _DH:_DH + 1] + 1e-9))
    o = jnp.concatenate(heads, axis=1)
    o = jnp.dot(o, wo, preferred_element_type=jnp.float32)

    def _ln(y, g, b):
        mu = jnp.mean(y, axis=1, keepdims=True)
        yc = y - mu
        var = jnp.mean(yc * yc, axis=1, keepdims=True)
        return yc / jnp.sqrt(var + 1e-5) * g + b

    x = _ln(x + o, g1, bg1)
    f = jnp.dot(
        jax.nn.gelu(jnp.dot(x, w1, preferred_element_type=jnp.float32) + b1),
        w2, preferred_element_type=jnp.float32) + b2
    return _ln(x + f, g2, bg2)


def _model_kernel(xe_ref, pw_ref, pb_ref, pe_ref,
                  wq_ref, wk_ref, wv_ref, wo_ref, rw_ref, rb_ref,
                  g1_ref, bg1_ref, w1_ref, b1_ref, w2_ref, b2_ref,
                  g2_ref, bg2_ref, o_ref, mean_ref, std_ref):
    raw = xe_ref[0]

    crow = jax.lax.broadcasted_iota(jnp.int32, (_C, _L), 0)
    ccol = jax.lax.broadcasted_iota(jnp.int32, (_C, _L), 1)
    csel = (ccol // _N == crow).astype(jnp.float32)
    rsum = jnp.sum(raw, axis=1, keepdims=True)
    rsumsq = jnp.sum(raw * raw, axis=1, keepdims=True)
    seg = jnp.dot(csel, jnp.concatenate([rsum, rsumsq], axis=1),
                  preferred_element_type=jnp.float32) / _T
    mean_c = seg[:, 0:1]
    var_c = seg[:, 1:2] - mean_c * mean_c
    std_c = jnp.sqrt(var_c + 1e-5)
    cselT = jnp.transpose(csel)
    mt = jnp.dot(cselT, mean_c, preferred_element_type=jnp.float32)
    st = jnp.dot(cselT, std_c, preferred_element_type=jnp.float32)
    xn = (raw - mt) / st

    x = (jnp.dot(xn, pw_ref[...], preferred_element_type=jnp.float32)
         + pb_ref[...] + pe_ref[...])

    row = jax.lax.broadcasted_iota(jnp.int32, (_L, _L), 0)
    col = jax.lax.broadcasted_iota(jnp.int32, (_L, _L), 1)
    loga_sel = (col == row,
                (col % _N) == (row % _N),
                (col // _N) == (row // _N))

    for l in range(_EL):
        x = _layer_body(x, wq_ref[l], wk_ref[l], wv_ref[l], wo_ref[l],
                        rw_ref[l], rb_ref[l], g1_ref[l], bg1_ref[l],
                        w1_ref[l], b1_ref[l], w2_ref[l], b2_ref[l],
                        g2_ref[l], bg2_ref[l], loga_sel)

    o_ref[0] = x
    mean_ref[0] = jnp.transpose(mean_c)
    std_ref[0] = jnp.transpose(std_c)


def _head_kernel(x_ref, w_ref, b_ref, mean_ref, std_ref, o_ref):
    xh = x_ref[0]
    out = jnp.dot(xh, w_ref[...], preferred_element_type=jnp.float32) + b_ref[...]
    o_ref[0] = jnp.transpose(out) * std_ref[0] + mean_ref[0]


def kernel(x_enc, x_mark_enc, x_dec, x_mark_dec, patch_W, patch_b, Wq, Wk, Wv,
           Wo, router_W, router_b, ln1_g, ln1_b, W1, b1, W2, b2, ln2_g, ln2_b,
           head_W, head_b):
    f32 = jnp.float32
    pe = jnp.asarray(_pos_encoding())

    full = lambda shape: pl.BlockSpec(shape, lambda b: (0,) * len(shape))
    x, mean, std = pl.pallas_call(
        _model_kernel,
        grid=(_B,),
        in_specs=[pl.BlockSpec((1, _L, _P), lambda b: (b, 0, 0)),
                  full((_P, _D)), full((1, _D)), full((_L, _D)),
                  full((_EL, _D, _D)), full((_EL, _D, _D)),
                  full((_EL, _D, _D)), full((_EL, _D, _D)),
                  full((_EL, _D, 3)), full((_EL, 1, 3)),
                  full((_EL, 1, _D)), full((_EL, 1, _D)),
                  full((_EL, _D, _DF)), full((_EL, 1, _DF)),
                  full((_EL, _DF, _D)), full((_EL, 1, _D)),
                  full((_EL, 1, _D)), full((_EL, 1, _D))],
        out_specs=[pl.BlockSpec((1, _L, _D), lambda b: (b, 0, 0)),
                   pl.BlockSpec((1, 1, _C), lambda b: (b, 0, 0)),
                   pl.BlockSpec((1, 1, _C), lambda b: (b, 0, 0))],
        out_shape=[jax.ShapeDtypeStruct((_B, _L, _D), f32),
                   jax.ShapeDtypeStruct((_B, 1, _C), f32),
                   jax.ShapeDtypeStruct((_B, 1, _C), f32)],
        compiler_params=pltpu.CompilerParams(
            dimension_semantics=("parallel",)),
    )(jnp.transpose(x_enc, (0, 2, 1)).reshape(_B, _L, _P),
      patch_W, patch_b.reshape(1, _D), pe,
      Wq, Wk, Wv, Wo, router_W, router_b.reshape(_EL, 1, 3),
      ln1_g.reshape(_EL, 1, _D), ln1_b.reshape(_EL, 1, _D),
      W1, b1.reshape(_EL, 1, _DF), W2, b2.reshape(_EL, 1, _D),
      ln2_g.reshape(_EL, 1, _D), ln2_b.reshape(_EL, 1, _D))

    xh = x.reshape(_B, _C, _N * _D)
    out = pl.pallas_call(
        _head_kernel,
        grid=(_B,),
        in_specs=[pl.BlockSpec((1, _C, _N * _D), lambda b: (b, 0, 0)),
                  full((_N * _D, _PRED)), full((1, _PRED)),
                  pl.BlockSpec((1, 1, _C), lambda b: (b, 0, 0)),
                  pl.BlockSpec((1, 1, _C), lambda b: (b, 0, 0))],
        out_specs=pl.BlockSpec((1, _PRED, _C), lambda b: (b, 0, 0)),
        out_shape=jax.ShapeDtypeStruct((_B, _PRED, _C), f32),
        compiler_params=pltpu.CompilerParams(
            dimension_semantics=("parallel",)),
    )(xh, head_W, head_b.reshape(1, _PRED), mean, std)

    return out


# speedup vs baseline: 1.3925x; 1.3925x over previous
import numpy as np
import jax
import jax.numpy as jnp
from jax.experimental import pallas as pl
from jax.experimental.pallas import tpu as pltpu

_B, _T, _C, _P, _D, _DF, _H, _DH = 8, 1024, 8, 16, 128, 256, 8, 16
_EL, _N, _L, _PRED, _TOPP = 2, 64, 512, 96, 0.5


def _pos_encoding_t():
    pos = np.arange(_L)[:, None].astype(np.float32)
    div = np.exp(np.arange(0, _D, 2).astype(np.float32) * (-np.log(10000.0) / _D))
    pe = np.zeros((_L, _D), dtype=np.float32)
    pe[:, 0::2] = np.sin(pos * div)
    pe[:, 1::2] = np.cos(pos * div)
    return pe.T.copy()


def _dot0(w, y):
    return jax.lax.dot_general(w, y, (((0,), (0,)), ((), ())),
                               preferred_element_type=jnp.float32)


def _layer_body(xT, wq, wk, wv, wo, rw, rbT, g1T, bg1T, w1, b1T, w2, b2T,
                g2T, bg2T, sel):
    rlogT = _dot0(rw, xT) + rbT
    l0, l1, l2 = rlogT[0:1], rlogT[1:2], rlogT[2:3]
    m = jnp.maximum(jnp.maximum(l0, l1), l2)
    e0, e1, e2 = jnp.exp(l0 - m), jnp.exp(l1 - m), jnp.exp(l2 - m)
    s = e0 + e1 + e2
    p0, p1, p2 = e0 / s, e1 / s, e2 / s
    cb0 = p1 * (p1 > p0) + p2 * (p2 > p0)
    cb1 = p0 * (p0 >= p1) + p2 * (p2 > p1)
    cb2 = p0 * (p0 >= p2) + p1 * (p1 >= p2)
    w0 = p0 * (cb0 < _TOPP)
    w1_ = p1 * (cb1 < _TOPP)
    w2_ = p2 * (cb2 < _TOPP)
    ws = w0 + w1_ + w2_ + 1e-9
    lw0 = jnp.log(w0 / ws)
    lw1 = jnp.log(w1_ / ws)
    lw2 = jnp.log(w2_ / ws)

    diag, same_s, same_t = sel
    logaT = jnp.where(diag, 0.0, jnp.where(same_s, lw0,
                      jnp.where(same_t, lw1, lw2)))

    qT = _dot0(wq, xT) * (1.0 / (_DH ** 0.5))
    kT = _dot0(wk, xT)
    vT = _dot0(wv, xT)
    ones_row = jnp.ones((1, _L), jnp.float32)

    headsT = []
    for h in range(_H):
        sl = slice(h * _DH, (h + 1) * _DH)
        sT = _dot0(kT[sl], qT[sl]) + logaT
        mh = jnp.max(sT, axis=0, keepdims=True)
        aeT = jnp.exp(sT - mh)
        vheT = jnp.concatenate([vT[sl], ones_row], axis=0)
        nzT = jnp.dot(vheT, aeT, preferred_element_type=jnp.float32)
        headsT.append(nzT[:_DH] / (nzT[_DH:_DH + 1] + 1e-9))
    oT = jnp.concatenate(headsT, axis=0)
    oT = _dot0(wo, oT)

    def _lnT(y, g, b):
        mu = jnp.mean(y, axis=0, keepdims=True)
        yc = y - mu
        var = jnp.mean(yc * yc, axis=0, keepdims=True)
        return yc / jnp.sqrt(var + 1e-5) * g + b

    xT = _lnT(xT + oT, g1T, bg1T)
    f = _dot0(w2, jax.nn.gelu(_dot0(w1, xT) + b1T)) + b2T
    return _lnT(xT + f, g2T, bg2T)


def _model_kernel(pt_ref, pw_ref, pbT_ref, peT_ref,
                  wq_ref, wk_ref, wv_ref, wo_ref, rw_ref, rbT_ref,
                  g1_ref, bg1_ref, w1_ref, b1_ref, w2_ref, b2_ref,
                  g2_ref, bg2_ref, o_ref, mean_ref, std_ref):
    ptT = pt_ref[0]

    crow = jax.lax.broadcasted_iota(jnp.int32, (_C, _L), 0)
    ccol = jax.lax.broadcasted_iota(jnp.int32, (_C, _L), 1)
    csel = (ccol // _N == crow).astype(jnp.float32)
    s12 = jnp.concatenate([jnp.sum(ptT, axis=0, keepdims=True),
                           jnp.sum(ptT * ptT, axis=0, keepdims=True)], axis=0)
    seg = jax.lax.dot_general(s12, csel, (((1,), (1,)), ((), ())),
                              preferred_element_type=jnp.float32) / _T
    mean_r = seg[0:1]
    var_r = seg[1:2] - mean_r * mean_r
    std_r = jnp.sqrt(var_r + 1e-5)
    mean_ref[0] = mean_r
    std_ref[0] = std_r
    mt = jnp.dot(mean_r, csel, preferred_element_type=jnp.float32)
    st = jnp.dot(std_r, csel, preferred_element_type=jnp.float32)
    xnT = (ptT - mt) / st

    xT = _dot0(pw_ref[...], xnT) + pbT_ref[...] + peT_ref[...]

    row = jax.lax.broadcasted_iota(jnp.int32, (_L, _L), 0)
    col = jax.lax.broadcasted_iota(jnp.int32, (_L, _L), 1)
    sel = (col == row,
           (col % _N) == (row % _N),
           (col // _N) == (row // _N))

    for l in range(_EL):
        xT = _layer_body(xT, wq_ref[l], wk_ref[l], wv_ref[l], wo_ref[l],
                         rw_ref[l], rbT_ref[l], g1_ref[l], bg1_ref[l],
                         w1_ref[l], b1_ref[l], w2_ref[l], b2_ref[l],
                         g2_ref[l], bg2_ref[l], sel)
    o_ref[0] = xT


def _head_kernel(x_ref, w_ref, b_ref, mean_ref, std_ref, o_ref):
    xh = x_ref[0]
    out = jnp.dot(xh, w_ref[...], preferred_element_type=jnp.float32) + b_ref[...]
    o_ref[0] = (out * jnp.transpose(std_ref[0])
                + jnp.transpose(mean_ref[0]))


def kernel(x_enc, x_mark_enc, x_dec, x_mark_dec, patch_W, patch_b, Wq, Wk, Wv,
           Wo, router_W, router_b, ln1_g, ln1_b, W1, b1, W2, b2, ln2_g, ln2_b,
           head_W, head_b):
    f32 = jnp.float32
    peT = jnp.asarray(_pos_encoding_t())

    patchesT = jnp.transpose(x_enc.reshape(_B, _N, _P, _C),
                             (0, 2, 3, 1)).reshape(_B, _P, _L)

    full = lambda shape: pl.BlockSpec(shape, lambda b: (0,) * len(shape))
    xT, mean, std = pl.pallas_call(
        _model_kernel,
        grid=(_B,),
        in_specs=[pl.BlockSpec((1, _P, _L), lambda b: (b, 0, 0)),
                  full((_P, _D)), full((_D, 1)), full((_D, _L)),
                  full((_EL, _D, _D)), full((_EL, _D, _D)),
                  full((_EL, _D, _D)), full((_EL, _D, _D)),
                  full((_EL, _D, 3)), full((_EL, 3, 1)),
                  full((_EL, _D, 1)), full((_EL, _D, 1)),
                  full((_EL, _D, _DF)), full((_EL, _DF, 1)),
                  full((_EL, _DF, _D)), full((_EL, _D, 1)),
                  full((_EL, _D, 1)), full((_EL, _D, 1))],
        out_specs=[pl.BlockSpec((1, _D, _L), lambda b: (b, 0, 0)),
                   pl.BlockSpec((1, 1, _C), lambda b: (b, 0, 0)),
                   pl.BlockSpec((1, 1, _C), lambda b: (b, 0, 0))],
        out_shape=[jax.ShapeDtypeStruct((_B, _D, _L), f32),
                   jax.ShapeDtypeStruct((_B, 1, _C), f32),
                   jax.ShapeDtypeStruct((_B, 1, _C), f32)],
        compiler_params=pltpu.CompilerParams(
            dimension_semantics=("parallel",)),
    )(patchesT, patch_W, patch_b.reshape(_D, 1), peT,
      Wq, Wk, Wv, Wo, router_W, router_b.reshape(_EL, 3, 1),
      ln1_g.reshape(_EL, _D, 1), ln1_b.reshape(_EL, _D, 1),
      W1, b1.reshape(_EL, _DF, 1), W2, b2.reshape(_EL, _D, 1),
      ln2_g.reshape(_EL, _D, 1), ln2_b.reshape(_EL, _D, 1))

    xh = jnp.transpose(xT, (0, 2, 1)).reshape(_B, _C, _N * _D)
    out = pl.pallas_call(
        _head_kernel,
        grid=(_B,),
        in_specs=[pl.BlockSpec((1, _C, _N * _D), lambda b: (b, 0, 0)),
                  full((_N * _D, _PRED)), full((1, _PRED)),
                  pl.BlockSpec((1, 1, _C), lambda b: (b, 0, 0)),
                  pl.BlockSpec((1, 1, _C), lambda b: (b, 0, 0))],
        out_specs=pl.BlockSpec((1, _C, _PRED), lambda b: (b, 0, 0)),
        out_shape=jax.ShapeDtypeStruct((_B, _C, _PRED), f32),
        compiler_params=pltpu.CompilerParams(
            dimension_semantics=("parallel",)),
    )(xh, head_W, head_b.reshape(1, _PRED), mean, std)

    return jnp.transpose(out, (0, 2, 1))


# speedup vs baseline: 1.3974x; 1.0035x over previous
import numpy as np
import jax
import jax.numpy as jnp
from jax.experimental import pallas as pl
from jax.experimental.pallas import tpu as pltpu

_B, _T, _C, _P, _D, _DF, _H, _DH = 8, 1024, 8, 16, 128, 256, 8, 16
_EL, _N, _L, _PRED, _TOPP = 2, 64, 512, 96, 0.5
_BPC = 1


def _pos_encoding_t():
    pos = np.arange(_L)[:, None].astype(np.float32)
    div = np.exp(np.arange(0, _D, 2).astype(np.float32) * (-np.log(10000.0) / _D))
    pe = np.zeros((_L, _D), dtype=np.float32)
    pe[:, 0::2] = np.sin(pos * div)
    pe[:, 1::2] = np.cos(pos * div)
    return pe.T.copy()


def _dot0(w, y):
    return jax.lax.dot_general(w, y, (((0,), (0,)), ((), ())),
                               preferred_element_type=jnp.float32)


def _layer_body(xT, wq, wk, wv, wo, rw, rbT, g1T, bg1T, w1, b1T, w2, b2T,
                g2T, bg2T, sel):
    rlogT = _dot0(rw, xT) + rbT
    l0, l1, l2 = rlogT[0:1], rlogT[1:2], rlogT[2:3]
    m = jnp.maximum(jnp.maximum(l0, l1), l2)
    e0, e1, e2 = jnp.exp(l0 - m), jnp.exp(l1 - m), jnp.exp(l2 - m)
    s = e0 + e1 + e2
    p0, p1, p2 = e0 / s, e1 / s, e2 / s
    cb0 = p1 * (p1 > p0) + p2 * (p2 > p0)
    cb1 = p0 * (p0 >= p1) + p2 * (p2 > p1)
    cb2 = p0 * (p0 >= p2) + p1 * (p1 >= p2)
    w0 = p0 * (cb0 < _TOPP)
    w1_ = p1 * (cb1 < _TOPP)
    w2_ = p2 * (cb2 < _TOPP)
    ws = w0 + w1_ + w2_ + 1e-9
    lw0 = jnp.log(w0 / ws)
    lw1 = jnp.log(w1_ / ws)
    lw2 = jnp.log(w2_ / ws)

    diag, same_s, same_t = sel
    logaT = jnp.where(diag, 0.0, jnp.where(same_s, lw0,
                      jnp.where(same_t, lw1, lw2)))

    qT = _dot0(wq, xT) * (1.0 / (_DH ** 0.5))
    kT = _dot0(wk, xT)
    vT = _dot0(wv, xT)
    ones_row = jnp.ones((1, _L), jnp.float32)

    headsT = []
    for h in range(_H):
        sl = slice(h * _DH, (h + 1) * _DH)
        sT = _dot0(kT[sl], qT[sl]) + logaT
        mh = jnp.max(sT, axis=0, keepdims=True)
        aeT = jnp.exp(sT - mh)
        vheT = jnp.concatenate([vT[sl], ones_row], axis=0)
        nzT = jnp.dot(vheT, aeT, preferred_element_type=jnp.float32)
        headsT.append(nzT[:_DH] / (nzT[_DH:_DH + 1] + 1e-9))
    oT = jnp.concatenate(headsT, axis=0)
    oT = _dot0(wo, oT)

    def _lnT(y, g, b):
        mu = jnp.mean(y, axis=0, keepdims=True)
        yc = y - mu
        var = jnp.mean(yc * yc, axis=0, keepdims=True)
        return yc / jnp.sqrt(var + 1e-5) * g + b

    xT = _lnT(xT + oT, g1T, bg1T)
    f = _dot0(w2, jax.nn.gelu(_dot0(w1, xT) + b1T)) + b2T
    return _lnT(xT + f, g2T, bg2T)


def _model_kernel(pt_ref, pw_ref, pbT_ref, peT_ref,
                  wq_ref, wk_ref, wv_ref, wo_ref, rw_ref, rbT_ref,
                  g1_ref, bg1_ref, w1_ref, b1_ref, w2_ref, b2_ref,
                  g2_ref, bg2_ref, o_ref, mean_ref, std_ref):
    crow = jax.lax.broadcasted_iota(jnp.int32, (_C, _L), 0)
    ccol = jax.lax.broadcasted_iota(jnp.int32, (_C, _L), 1)
    csel = (ccol // _N == crow).astype(jnp.float32)
    row = jax.lax.broadcasted_iota(jnp.int32, (_L, _L), 0)
    col = jax.lax.broadcasted_iota(jnp.int32, (_L, _L), 1)
    sel = (col == row,
           (col % _N) == (row % _N),
           (col // _N) == (row // _N))

    for i in range(_BPC):
        ptT = pt_ref[i]

        s12 = jnp.concatenate([jnp.sum(ptT, axis=0, keepdims=True),
                               jnp.sum(ptT * ptT, axis=0, keepdims=True)],
                              axis=0)
        seg = jax.lax.dot_general(s12, csel, (((1,), (1,)), ((), ())),
                                  preferred_element_type=jnp.float32) / _T
        mean_r = seg[0:1]
        var_r = seg[1:2] - mean_r * mean_r
        std_r = jnp.sqrt(var_r + 1e-5)
        mean_ref[i] = mean_r
        std_ref[i] = std_r
        mt = jnp.dot(mean_r, csel, preferred_element_type=jnp.float32)
        st = jnp.dot(std_r, csel, preferred_element_type=jnp.float32)
        xnT = (ptT - mt) / st

        xT = _dot0(pw_ref[...], xnT) + pbT_ref[...] + peT_ref[...]

        for l in range(_EL):
            xT = _layer_body(xT, wq_ref[l], wk_ref[l], wv_ref[l], wo_ref[l],
                             rw_ref[l], rbT_ref[l], g1_ref[l], bg1_ref[l],
                             w1_ref[l], b1_ref[l], w2_ref[l], b2_ref[l],
                             g2_ref[l], bg2_ref[l], sel)
        o_ref[i] = xT


def _head_kernel(x_ref, w_ref, b_ref, mean_ref, std_ref, o_ref):
    xh = x_ref[0]
    out = jnp.dot(xh, w_ref[...], preferred_element_type=jnp.float32) + b_ref[...]
    o_ref[0] = (out * jnp.transpose(std_ref[0])
                + jnp.transpose(mean_ref[0]))


def kernel(x_enc, x_mark_enc, x_dec, x_mark_dec, patch_W, patch_b, Wq, Wk, Wv,
           Wo, router_W, router_b, ln1_g, ln1_b, W1, b1, W2, b2, ln2_g, ln2_b,
           head_W, head_b):
    f32 = jnp.float32
    peT = jnp.asarray(_pos_encoding_t())

    patchesT = jnp.transpose(x_enc.reshape(_B, _N, _P, _C),
                             (0, 2, 3, 1)).reshape(_B, _P, _L)

    full = lambda shape: pl.BlockSpec(shape, lambda b: (0,) * len(shape))
    xT, mean, std = pl.pallas_call(
        _model_kernel,
        grid=(_B,),
        in_specs=[pl.BlockSpec((1, _P, _L), lambda b: (b, 0, 0)),
                  full((_P, _D)), full((_D, 1)), full((_D, _L)),
                  full((_EL, _D, _D)), full((_EL, _D, _D)),
                  full((_EL, _D, _D)), full((_EL, _D, _D)),
                  full((_EL, _D, 3)), full((_EL, 3, 1)),
                  full((_EL, _D, 1)), full((_EL, _D, 1)),
                  full((_EL, _D, _DF)), full((_EL, _DF, 1)),
                  full((_EL, _DF, _D)), full((_EL, _D, 1)),
                  full((_EL, _D, 1)), full((_EL, _D, 1))],
        out_specs=[pl.BlockSpec((1, _D, _L), lambda b: (b, 0, 0)),
                   pl.BlockSpec((1, 1, _C), lambda b: (b, 0, 0)),
                   pl.BlockSpec((1, 1, _C), lambda b: (b, 0, 0))],
        out_shape=[jax.ShapeDtypeStruct((_B, _D, _L), f32),
                   jax.ShapeDtypeStruct((_B, 1, _C), f32),
                   jax.ShapeDtypeStruct((_B, 1, _C), f32)],
        compiler_params=pltpu.CompilerParams(
            dimension_semantics=("parallel",)),
    )(patchesT, patch_W, patch_b.reshape(_D, 1), peT,
      Wq, Wk, Wv, Wo, router_W, router_b.reshape(_EL, 3, 1),
      ln1_g.reshape(_EL, _D, 1), ln1_b.reshape(_EL, _D, 1),
      W1, b1.reshape(_EL, _DF, 1), W2, b2.reshape(_EL, _D, 1),
      ln2_g.reshape(_EL, _D, 1), ln2_b.reshape(_EL, _D, 1))

    xh = jnp.transpose(xT, (0, 2, 1)).reshape(_B, _C, _N * _D)
    out = pl.pallas_call(
        _head_kernel,
        grid=(_B,),
        in_specs=[pl.BlockSpec((1, _C, _N * _D), lambda b: (b, 0, 0)),
                  full((_N * _D, _PRED)), full((1, _PRED)),
                  pl.BlockSpec((1, 1, _C), lambda b: (b, 0, 0)),
                  pl.BlockSpec((1, 1, _C), lambda b: (b, 0, 0))],
        out_specs=pl.BlockSpec((1, _C, _PRED), lambda b: (b, 0, 0)),
        out_shape=jax.ShapeDtypeStruct((_B, _C, _PRED), f32),
        compiler_params=pltpu.CompilerParams(
            dimension_semantics=("parallel",)),
    )(xh, head_W, head_b.reshape(1, _PRED), mean, std)

    return jnp.transpose(out, (0, 2, 1))


# speedup vs baseline: 1.4048x; 1.0053x over previous
import numpy as np
import jax
import jax.numpy as jnp
from jax.experimental import pallas as pl
from jax.experimental.pallas import tpu as pltpu

_B, _T, _C, _P, _D, _DF, _H, _DH = 8, 1024, 8, 16, 128, 256, 8, 16
_EL, _N, _L, _PRED, _TOPP = 2, 64, 512, 96, 0.5
_BPC = 2


def _pos_encoding_t():
    pos = np.arange(_L)[:, None].astype(np.float32)
    div = np.exp(np.arange(0, _D, 2).astype(np.float32) * (-np.log(10000.0) / _D))
    pe = np.zeros((_L, _D), dtype=np.float32)
    pe[:, 0::2] = np.sin(pos * div)
    pe[:, 1::2] = np.cos(pos * div)
    return pe.T.copy()


def _dot0(w, y):
    return jax.lax.dot_general(w, y, (((0,), (0,)), ((), ())),
                               preferred_element_type=jnp.float32)


def _layer_body(xT, wq, wk, wv, wo, rw, rbT, g1T, bg1T, w1, b1T, w2, b2T,
                g2T, bg2T, sel):
    rlogT = _dot0(rw, xT) + rbT
    l0, l1, l2 = rlogT[0:1], rlogT[1:2], rlogT[2:3]
    m = jnp.maximum(jnp.maximum(l0, l1), l2)
    e0, e1, e2 = jnp.exp(l0 - m), jnp.exp(l1 - m), jnp.exp(l2 - m)
    s = e0 + e1 + e2
    p0, p1, p2 = e0 / s, e1 / s, e2 / s
    cb0 = p1 * (p1 > p0) + p2 * (p2 > p0)
    cb1 = p0 * (p0 >= p1) + p2 * (p2 > p1)
    cb2 = p0 * (p0 >= p2) + p1 * (p1 >= p2)
    w0 = p0 * (cb0 < _TOPP)
    w1_ = p1 * (cb1 < _TOPP)
    w2_ = p2 * (cb2 < _TOPP)
    ws = w0 + w1_ + w2_ + 1e-9
    lw0 = jnp.log(w0 / ws)
    lw1 = jnp.log(w1_ / ws)
    lw2 = jnp.log(w2_ / ws)

    diag, same_s, same_t = sel
    logaT = jnp.where(diag, 0.0, jnp.where(same_s, lw0,
                      jnp.where(same_t, lw1, lw2)))

    qT = _dot0(wq, xT) * (1.0 / (_DH ** 0.5))
    kT = _dot0(wk, xT)
    vT = _dot0(wv, xT)
    ones_row = jnp.ones((1, _L), jnp.float32)

    headsT = []
    for h in range(_H):
        sl = slice(h * _DH, (h + 1) * _DH)
        sT = _dot0(kT[sl], qT[sl]) + logaT
        mh = jnp.max(sT, axis=0, keepdims=True)
        aeT = jnp.exp(sT - mh)
        vheT = jnp.concatenate([vT[sl], ones_row], axis=0)
        nzT = jnp.dot(vheT, aeT, preferred_element_type=jnp.float32)
        headsT.append(nzT[:_DH] / (nzT[_DH:_DH + 1] + 1e-9))
    oT = jnp.concatenate(headsT, axis=0)
    oT = _dot0(wo, oT)

    def _lnT(y, g, b):
        mu = jnp.mean(y, axis=0, keepdims=True)
        yc = y - mu
        var = jnp.mean(yc * yc, axis=0, keepdims=True)
        return yc / jnp.sqrt(var + 1e-5) * g + b

    xT = _lnT(xT + oT, g1T, bg1T)
    f = _dot0(w2, jax.nn.gelu(_dot0(w1, xT) + b1T)) + b2T
    return _lnT(xT + f, g2T, bg2T)


def _model_kernel(pt_ref, pw_ref, pbT_ref, peT_ref,
                  wq_ref, wk_ref, wv_ref, wo_ref, rw_ref, rbT_ref,
                  g1_ref, bg1_ref, w1_ref, b1_ref, w2_ref, b2_ref,
                  g2_ref, bg2_ref, o_ref, mean_ref, std_ref):
    crow = jax.lax.broadcasted_iota(jnp.int32, (_C, _L), 0)
    ccol = jax.lax.broadcasted_iota(jnp.int32, (_C, _L), 1)
    csel = (ccol // _N == crow).astype(jnp.float32)
    row = jax.lax.broadcasted_iota(jnp.int32, (_L, _L), 0)
    col = jax.lax.broadcasted_iota(jnp.int32, (_L, _L), 1)
    sel = (col == row,
           (col % _N) == (row % _N),
           (col // _N) == (row // _N))

    for i in range(_BPC):
        ptT = pt_ref[i]

        s12 = jnp.concatenate([jnp.sum(ptT, axis=0, keepdims=True),
                               jnp.sum(ptT * ptT, axis=0, keepdims=True)],
                              axis=0)
        seg = jax.lax.dot_general(s12, csel, (((1,), (1,)), ((), ())),
                                  preferred_element_type=jnp.float32) / _T
        mean_r = seg[0:1]
        var_r = seg[1:2] - mean_r * mean_r
        std_r = jnp.sqrt(var_r + 1e-5)
        mean_ref[i] = mean_r
        std_ref[i] = std_r
        mt = jnp.dot(mean_r, csel, preferred_element_type=jnp.float32)
        st = jnp.dot(std_r, csel, preferred_element_type=jnp.float32)
        xnT = (ptT - mt) / st

        xT = _dot0(pw_ref[...], xnT) + pbT_ref[...] + peT_ref[...]

        for l in range(_EL):
            xT = _layer_body(xT, wq_ref[l], wk_ref[l], wv_ref[l], wo_ref[l],
                             rw_ref[l], rbT_ref[l], g1_ref[l], bg1_ref[l],
                             w1_ref[l], b1_ref[l], w2_ref[l], b2_ref[l],
                             g2_ref[l], bg2_ref[l], sel)
        o_ref[i] = xT


def _head_kernel(x_ref, w_ref, b_ref, mean_ref, std_ref, o_ref):
    xh = x_ref[0]
    out = jnp.dot(xh, w_ref[...], preferred_element_type=jnp.float32) + b_ref[...]
    o_ref[0] = (out * jnp.transpose(std_ref[0])
                + jnp.transpose(mean_ref[0]))


def kernel(x_enc, x_mark_enc, x_dec, x_mark_dec, patch_W, patch_b, Wq, Wk, Wv,
           Wo, router_W, router_b, ln1_g, ln1_b, W1, b1, W2, b2, ln2_g, ln2_b,
           head_W, head_b):
    f32 = jnp.float32
    peT = jnp.asarray(_pos_encoding_t())

    patchesT = jnp.transpose(x_enc.reshape(_B, _N, _P, _C),
                             (0, 2, 3, 1)).reshape(_B, _P, _L)

    full = lambda shape: pl.BlockSpec(shape, lambda b: (0,) * len(shape))
    xT, mean, std = pl.pallas_call(
        _model_kernel,
        grid=(_B // _BPC,),
        in_specs=[pl.BlockSpec((_BPC, _P, _L), lambda b: (b, 0, 0)),
                  full((_P, _D)), full((_D, 1)), full((_D, _L)),
                  full((_EL, _D, _D)), full((_EL, _D, _D)),
                  full((_EL, _D, _D)), full((_EL, _D, _D)),
                  full((_EL, _D, 3)), full((_EL, 3, 1)),
                  full((_EL, _D, 1)), full((_EL, _D, 1)),
                  full((_EL, _D, _DF)), full((_EL, _DF, 1)),
                  full((_EL, _DF, _D)), full((_EL, _D, 1)),
                  full((_EL, _D, 1)), full((_EL, _D, 1))],
        out_specs=[pl.BlockSpec((_BPC, _D, _L), lambda b: (b, 0, 0)),
                   pl.BlockSpec((_BPC, 1, _C), lambda b: (b, 0, 0)),
                   pl.BlockSpec((_BPC, 1, _C), lambda b: (b, 0, 0))],
        out_shape=[jax.ShapeDtypeStruct((_B, _D, _L), f32),
                   jax.ShapeDtypeStruct((_B, 1, _C), f32),
                   jax.ShapeDtypeStruct((_B, 1, _C), f32)],
        compiler_params=pltpu.CompilerParams(
            dimension_semantics=("parallel",)),
    )(patchesT, patch_W, patch_b.reshape(_D, 1), peT,
      Wq, Wk, Wv, Wo, router_W, router_b.reshape(_EL, 3, 1),
      ln1_g.reshape(_EL, _D, 1), ln1_b.reshape(_EL, _D, 1),
      W1, b1.reshape(_EL, _DF, 1), W2, b2.reshape(_EL, _D, 1),
      ln2_g.reshape(_EL, _D, 1), ln2_b.reshape(_EL, _D, 1))

    xh = jnp.transpose(xT, (0, 2, 1)).reshape(_B, _C, _N * _D)
    out = pl.pallas_call(
        _head_kernel,
        grid=(_B,),
        in_specs=[pl.BlockSpec((1, _C, _N * _D), lambda b: (b, 0, 0)),
                  full((_N * _D, _PRED)), full((1, _PRED)),
                  pl.BlockSpec((1, 1, _C), lambda b: (b, 0, 0)),
                  pl.BlockSpec((1, 1, _C), lambda b: (b, 0, 0))],
        out_specs=pl.BlockSpec((1, _C, _PRED), lambda b: (b, 0, 0)),
        out_shape=jax.ShapeDtypeStruct((_B, _C, _PRED), f32),
        compiler_params=pltpu.CompilerParams(
            dimension_semantics=("parallel",)),
    )(xh, head_W, head_b.reshape(1, _PRED), mean, std)

    return jnp.transpose(out, (0, 2, 1))


# speedup vs baseline: 1.4179x; 1.0093x over previous
import numpy as np
import jax
import jax.numpy as jnp
from jax.experimental import pallas as pl
from jax.experimental.pallas import tpu as pltpu

_B, _T, _C, _P, _D, _DF, _H, _DH = 8, 1024, 8, 16, 128, 256, 8, 16
_EL, _N, _L, _PRED, _TOPP = 2, 64, 512, 96, 0.5
_BPC = 2


def _pos_encoding_t():
    pos = np.arange(_L)[:, None].astype(np.float32)
    div = np.exp(np.arange(0, _D, 2).astype(np.float32) * (-np.log(10000.0) / _D))
    pe = np.zeros((_L, _D), dtype=np.float32)
    pe[:, 0::2] = np.sin(pos * div)
    pe[:, 1::2] = np.cos(pos * div)
    return pe.T.copy()


def _dot0(w, y):
    return jax.lax.dot_general(w, y, (((0,), (0,)), ((), ())),
                               preferred_element_type=jnp.float32)


def _layer_body(xT, wq, wk, wv, wo, rw, rbT, g1T, bg1T, w1, b1T, w2, b2T,
                g2T, bg2T, sel):
    rlogT = _dot0(rw, xT) + rbT
    l0, l1, l2 = rlogT[0:1], rlogT[1:2], rlogT[2:3]
    m = jnp.maximum(jnp.maximum(l0, l1), l2)
    e0, e1, e2 = jnp.exp(l0 - m), jnp.exp(l1 - m), jnp.exp(l2 - m)
    s = e0 + e1 + e2
    p0, p1, p2 = e0 / s, e1 / s, e2 / s
    cb0 = p1 * (p1 > p0) + p2 * (p2 > p0)
    cb1 = p0 * (p0 >= p1) + p2 * (p2 > p1)
    cb2 = p0 * (p0 >= p2) + p1 * (p1 >= p2)
    w0 = p0 * (cb0 < _TOPP)
    w1_ = p1 * (cb1 < _TOPP)
    w2_ = p2 * (cb2 < _TOPP)
    ws = w0 + w1_ + w2_ + 1e-9
    lw0 = jnp.log(w0 / ws)
    lw1 = jnp.log(w1_ / ws)
    lw2 = jnp.log(w2_ / ws)

    diag, same_s, same_t = sel
    logaT = jnp.where(diag, 0.0, jnp.where(same_s, lw0,
                      jnp.where(same_t, lw1, lw2)))

    qT = _dot0(wq, xT) * (1.0 / (_DH ** 0.5))
    kT = _dot0(wk, xT)
    vT = _dot0(wv, xT)
    ones_row = jnp.ones((1, _L), jnp.float32)

    headsT = []
    for h in range(_H):
        sl = slice(h * _DH, (h + 1) * _DH)
        sT = _dot0(kT[sl], qT[sl]) + logaT
        mh = jnp.max(sT, axis=0, keepdims=True)
        aeT = jnp.exp(sT - mh)
        vheT = jnp.concatenate([vT[sl], ones_row], axis=0)
        nzT = jnp.dot(vheT, aeT, preferred_element_type=jnp.float32)
        headsT.append(nzT[:_DH] / (nzT[_DH:_DH + 1] + 1e-9))
    oT = jnp.concatenate(headsT, axis=0)
    oT = _dot0(wo, oT)

    def _lnT(y, g, b):
        mu = jnp.mean(y, axis=0, keepdims=True)
        yc = y - mu
        var = jnp.mean(yc * yc, axis=0, keepdims=True)
        return yc / jnp.sqrt(var + 1e-5) * g + b

    xT = _lnT(xT + oT, g1T, bg1T)
    f = _dot0(w2, jax.nn.gelu(_dot0(w1, xT) + b1T)) + b2T
    return _lnT(xT + f, g2T, bg2T)


def _model_kernel(pt_ref, pw_ref, pbT_ref, peT_ref,
                  wq_ref, wk_ref, wv_ref, wo_ref, rw_ref, rbT_ref,
                  g1_ref, bg1_ref, w1_ref, b1_ref, w2_ref, b2_ref,
                  g2_ref, bg2_ref, o_ref, mean_ref, std_ref):
    crow = jax.lax.broadcasted_iota(jnp.int32, (_C, _L), 0)
    ccol = jax.lax.broadcasted_iota(jnp.int32, (_C, _L), 1)
    csel = (ccol // _N == crow).astype(jnp.float32)
    row = jax.lax.broadcasted_iota(jnp.int32, (_L, _L), 0)
    col = jax.lax.broadcasted_iota(jnp.int32, (_L, _L), 1)
    sel = (col == row,
           (col % _N) == (row % _N),
           (col // _N) == (row // _N))

    for i in range(_BPC):
        ptT = pt_ref[i]

        s12 = jnp.concatenate([jnp.sum(ptT, axis=0, keepdims=True),
                               jnp.sum(ptT * ptT, axis=0, keepdims=True)],
                              axis=0)
        seg = jax.lax.dot_general(s12, csel, (((1,), (1,)), ((), ())),
                                  preferred_element_type=jnp.float32) / _T
        mean_r = seg[0:1]
        var_r = seg[1:2] - mean_r * mean_r
        std_r = jnp.sqrt(var_r + 1e-5)
        mean_ref[i] = mean_r
        std_ref[i] = std_r
        mt = jnp.dot(mean_r, csel, preferred_element_type=jnp.float32)
        st = jnp.dot(std_r, csel, preferred_element_type=jnp.float32)
        xnT = (ptT - mt) / st

        xT = _dot0(pw_ref[...], xnT) + pbT_ref[...] + peT_ref[...]

        for l in range(_EL):
            xT = _layer_body(xT, wq_ref[l], wk_ref[l], wv_ref[l], wo_ref[l],
                             rw_ref[l], rbT_ref[l], g1_ref[l], bg1_ref[l],
                             w1_ref[l], b1_ref[l], w2_ref[l], b2_ref[l],
                             g2_ref[l], bg2_ref[l], sel)
        o_ref[i] = jnp.transpose(xT)


def _head_kernel(x_ref, w_ref, b_ref, mean_ref, std_ref, o_ref):
    xh = x_ref[0]
    out = jnp.dot(xh, w_ref[...], preferred_element_type=jnp.float32) + b_ref[...]
    o_ref[0] = (out * jnp.transpose(std_ref[0])
                + jnp.transpose(mean_ref[0]))


def kernel(x_enc, x_mark_enc, x_dec, x_mark_dec, patch_W, patch_b, Wq, Wk, Wv,
           Wo, router_W, router_b, ln1_g, ln1_b, W1, b1, W2, b2, ln2_g, ln2_b,
           head_W, head_b):
    f32 = jnp.float32
    peT = jnp.asarray(_pos_encoding_t())

    patchesT = jnp.transpose(x_enc.reshape(_B, _N, _P, _C),
                             (0, 2, 3, 1)).reshape(_B, _P, _L)

    full = lambda shape: pl.BlockSpec(shape, lambda b: (0,) * len(shape))
    xT, mean, std = pl.pallas_call(
        _model_kernel,
        grid=(_B // _BPC,),
        in_specs=[pl.BlockSpec((_BPC, _P, _L), lambda b: (b, 0, 0)),
                  full((_P, _D)), full((_D, 1)), full((_D, _L)),
                  full((_EL, _D, _D)), full((_EL, _D, _D)),
                  full((_EL, _D, _D)), full((_EL, _D, _D)),
                  full((_EL, _D, 3)), full((_EL, 3, 1)),
                  full((_EL, _D, 1)), full((_EL, _D, 1)),
                  full((_EL, _D, _DF)), full((_EL, _DF, 1)),
                  full((_EL, _DF, _D)), full((_EL, _D, 1)),
                  full((_EL, _D, 1)), full((_EL, _D, 1))],
        out_specs=[pl.BlockSpec((_BPC, _L, _D), lambda b: (b, 0, 0)),
                   pl.BlockSpec((_BPC, 1, _C), lambda b: (b, 0, 0)),
                   pl.BlockSpec((_BPC, 1, _C), lambda b: (b, 0, 0))],
        out_shape=[jax.ShapeDtypeStruct((_B, _L, _D), f32),
                   jax.ShapeDtypeStruct((_B, 1, _C), f32),
                   jax.ShapeDtypeStruct((_B, 1, _C), f32)],
        compiler_params=pltpu.CompilerParams(
            dimension_semantics=("parallel",)),
    )(patchesT, patch_W, patch_b.reshape(_D, 1), peT,
      Wq, Wk, Wv, Wo, router_W, router_b.reshape(_EL, 3, 1),
      ln1_g.reshape(_EL, _D, 1), ln1_b.reshape(_EL, _D, 1),
      W1, b1.reshape(_EL, _DF, 1), W2, b2.reshape(_EL, _D, 1),
      ln2_g.reshape(_EL, _D, 1), ln2_b.reshape(_EL, _D, 1))

    xh = xT.reshape(_B, _C, _N * _D)
    out = pl.pallas_call(
        _head_kernel,
        grid=(_B,),
        in_specs=[pl.BlockSpec((1, _C, _N * _D), lambda b: (b, 0, 0)),
                  full((_N * _D, _PRED)), full((1, _PRED)),
                  pl.BlockSpec((1, 1, _C), lambda b: (b, 0, 0)),
                  pl.BlockSpec((1, 1, _C), lambda b: (b, 0, 0))],
        out_specs=pl.BlockSpec((1, _C, _PRED), lambda b: (b, 0, 0)),
        out_shape=jax.ShapeDtypeStruct((_B, _C, _PRED), f32),
        compiler_params=pltpu.CompilerParams(
            dimension_semantics=("parallel",)),
    )(xh, head_W, head_b.reshape(1, _PRED), mean, std)

    return jnp.transpose(out, (0, 2, 1))


# speedup vs baseline: 1.4404x; 1.0159x over previous
import numpy as np
import jax
import jax.numpy as jnp
from jax.experimental import pallas as pl
from jax.experimental.pallas import tpu as pltpu

_B, _T, _C, _P, _D, _DF, _H, _DH = 8, 1024, 8, 16, 128, 256, 8, 16
_EL, _N, _L, _PRED, _TOPP = 2, 64, 512, 96, 0.5
_BPC = 2


def _pos_encoding_t():
    pos = np.arange(_L)[:, None].astype(np.float32)
    div = np.exp(np.arange(0, _D, 2).astype(np.float32) * (-np.log(10000.0) / _D))
    pe = np.zeros((_L, _D), dtype=np.float32)
    pe[:, 0::2] = np.sin(pos * div)
    pe[:, 1::2] = np.cos(pos * div)
    peT = pe.T.copy()
    return peT.reshape(_D, _C, _N).transpose(0, 2, 1).reshape(_D, _L).copy()


def _dot0(w, y):
    return jax.lax.dot_general(w, y, (((0,), (0,)), ((), ())),
                               preferred_element_type=jnp.float32)


def _layer_body(xT, wq, wk, wv, wo, rw, rbT, g1T, bg1T, w1, b1T, w2, b2T,
                g2T, bg2T, sel):
    rlogT = _dot0(rw, xT) + rbT
    l0, l1, l2 = rlogT[0:1], rlogT[1:2], rlogT[2:3]
    m = jnp.maximum(jnp.maximum(l0, l1), l2)
    e0, e1, e2 = jnp.exp(l0 - m), jnp.exp(l1 - m), jnp.exp(l2 - m)
    s = e0 + e1 + e2
    p0, p1, p2 = e0 / s, e1 / s, e2 / s
    cb0 = p1 * (p1 > p0) + p2 * (p2 > p0)
    cb1 = p0 * (p0 >= p1) + p2 * (p2 > p1)
    cb2 = p0 * (p0 >= p2) + p1 * (p1 >= p2)
    w0 = p0 * (cb0 < _TOPP)
    w1_ = p1 * (cb1 < _TOPP)
    w2_ = p2 * (cb2 < _TOPP)
    ws = w0 + w1_ + w2_ + 1e-9
    lw0 = jnp.log(w0 / ws)
    lw1 = jnp.log(w1_ / ws)
    lw2 = jnp.log(w2_ / ws)

    diag, same_s, same_t = sel
    logaT = jnp.where(diag, 0.0, jnp.where(same_s, lw0,
                      jnp.where(same_t, lw1, lw2)))

    qT = _dot0(wq, xT) * (1.0 / (_DH ** 0.5))
    kT = _dot0(wk, xT)
    vT = _dot0(wv, xT)
    ones_row = jnp.ones((1, _L), jnp.float32)

    headsT = []
    for h in range(_H):
        sl = slice(h * _DH, (h + 1) * _DH)
        sT = _dot0(kT[sl], qT[sl]) + logaT
        mh = jnp.max(sT, axis=0, keepdims=True)
        aeT = jnp.exp(sT - mh)
        vheT = jnp.concatenate([vT[sl], ones_row], axis=0)
        nzT = jnp.dot(vheT, aeT, preferred_element_type=jnp.float32)
        headsT.append(nzT[:_DH] / (nzT[_DH:_DH + 1] + 1e-9))
    oT = jnp.concatenate(headsT, axis=0)
    oT = _dot0(wo, oT)

    def _lnT(y, g, b):
        mu = jnp.mean(y, axis=0, keepdims=True)
        yc = y - mu
        var = jnp.mean(yc * yc, axis=0, keepdims=True)
        return yc / jnp.sqrt(var + 1e-5) * g + b

    xT = _lnT(xT + oT, g1T, bg1T)
    f = _dot0(w2, jax.nn.gelu(_dot0(w1, xT) + b1T)) + b2T
    return _lnT(xT + f, g2T, bg2T)


def _model_kernel(pt_ref, pw_ref, pbT_ref, peT_ref,
                  wq_ref, wk_ref, wv_ref, wo_ref, rw_ref, rbT_ref,
                  g1_ref, bg1_ref, w1_ref, b1_ref, w2_ref, b2_ref,
                  g2_ref, bg2_ref, hw_ref, hb_ref, o_ref):
    crow = jax.lax.broadcasted_iota(jnp.int32, (_C, _L), 0)
    ccol = jax.lax.broadcasted_iota(jnp.int32, (_C, _L), 1)
    csel = (ccol % _C == crow).astype(jnp.float32)
    row = jax.lax.broadcasted_iota(jnp.int32, (_L, _L), 0)
    col = jax.lax.broadcasted_iota(jnp.int32, (_L, _L), 1)
    sel = (col == row,
           (col // _C) == (row // _C),
           (col % _C) == (row % _C))

    for i in range(_BPC):
        ptT = pt_ref[i]

        s12 = jnp.concatenate([jnp.sum(ptT, axis=0, keepdims=True),
                               jnp.sum(ptT * ptT, axis=0, keepdims=True)],
                              axis=0)
        seg = jax.lax.dot_general(s12, csel, (((1,), (1,)), ((), ())),
                                  preferred_element_type=jnp.float32) / _T
        mean_r = seg[0:1]
        var_r = seg[1:2] - mean_r * mean_r
        std_r = jnp.sqrt(var_r + 1e-5)
        mt = jnp.dot(mean_r, csel, preferred_element_type=jnp.float32)
        st = jnp.dot(std_r, csel, preferred_element_type=jnp.float32)
        xnT = (ptT - mt) / st

        xT = _dot0(pw_ref[...], xnT) + pbT_ref[...] + peT_ref[...]

        for l in range(_EL):
            xT = _layer_body(xT, wq_ref[l], wk_ref[l], wv_ref[l], wo_ref[l],
                             rw_ref[l], rbT_ref[l], g1_ref[l], bg1_ref[l],
                             w1_ref[l], b1_ref[l], w2_ref[l], b2_ref[l],
                             g2_ref[l], bg2_ref[l], sel)

        acc = jnp.broadcast_to(hb_ref[...], (_C, _PRED))
        for n in range(_N):
            acc = acc + _dot0(xT[:, n * _C:(n + 1) * _C], hw_ref[n])
        o_ref[i] = acc * jnp.transpose(std_r) + jnp.transpose(mean_r)


def kernel(x_enc, x_mark_enc, x_dec, x_mark_dec, patch_W, patch_b, Wq, Wk, Wv,
           Wo, router_W, router_b, ln1_g, ln1_b, W1, b1, W2, b2, ln2_g, ln2_b,
           head_W, head_b):
    f32 = jnp.float32
    peT = jnp.asarray(_pos_encoding_t())

    patchesT = jnp.transpose(x_enc.reshape(_B, _N, _P, _C),
                             (0, 2, 1, 3)).reshape(_B, _P, _L)

    full = lambda shape: pl.BlockSpec(shape, lambda b: (0,) * len(shape))
    out = pl.pallas_call(
        _model_kernel,
        grid=(_B // _BPC,),
        in_specs=[pl.BlockSpec((_BPC, _P, _L), lambda b: (b, 0, 0)),
                  full((_P, _D)), full((_D, 1)), full((_D, _L)),
                  full((_EL, _D, _D)), full((_EL, _D, _D)),
                  full((_EL, _D, _D)), full((_EL, _D, _D)),
                  full((_EL, _D, 3)), full((_EL, 3, 1)),
                  full((_EL, _D, 1)), full((_EL, _D, 1)),
                  full((_EL, _D, _DF)), full((_EL, _DF, 1)),
                  full((_EL, _DF, _D)), full((_EL, _D, 1)),
                  full((_EL, _D, 1)), full((_EL, _D, 1)),
                  full((_N, _D, _PRED)), full((1, _PRED))],
        out_specs=pl.BlockSpec((_BPC, _C, _PRED), lambda b: (b, 0, 0)),
        out_shape=jax.ShapeDtypeStruct((_B, _C, _PRED), f32),
        compiler_params=pltpu.CompilerParams(
            dimension_semantics=("parallel",)),
    )(patchesT, patch_W, patch_b.reshape(_D, 1), peT,
      Wq, Wk, Wv, Wo, router_W, router_b.reshape(_EL, 3, 1),
      ln1_g.reshape(_EL, _D, 1), ln1_b.reshape(_EL, _D, 1),
      W1, b1.reshape(_EL, _DF, 1), W2, b2.reshape(_EL, _D, 1),
      ln2_g.reshape(_EL, _D, 1), ln2_b.reshape(_EL, _D, 1),
      head_W.reshape(_N, _D, _PRED), head_b.reshape(1, _PRED))

    return jnp.transpose(out, (0, 2, 1))
